# Initial kernel scaffold; baseline (speedup 1.0000x reference)
import jax
import jax.numpy as jnp
from jax.experimental import pallas as pl


def kernel(x, emb):
    raise NotImplementedError("write your pallas kernel here")


# speedup vs baseline: 1.6564x; 1.6564x over previous
import jax
import jax.numpy as jnp
from jax.experimental import pallas as pl


def _add_body(x_ref, e_ref, o_ref):
    o_ref[...] = x_ref[...] + e_ref[...]


def kernel(x, emb):
    B, T, H = x.shape
    bt = 512

    return pl.pallas_call(
        _add_body,
        grid=(T // bt, B),
        in_specs=[
            pl.BlockSpec((1, bt, H), lambda t, b: (b, t, 0)),
            pl.BlockSpec((bt, H), lambda t, b: (t, 0)),
        ],
        out_specs=pl.BlockSpec((1, bt, H), lambda t, b: (b, t, 0)),
        out_shape=jax.ShapeDtypeStruct(x.shape, x.dtype),
    )(x, emb[:T])


# speedup vs baseline: 1.7350x; 1.0474x over previous
import jax
import jax.numpy as jnp
from jax.experimental import pallas as pl


def _add_body(x_ref, e_ref, o_ref):
    o_ref[...] = x_ref[...] + e_ref[...]


def kernel(x, emb):
    B, T, H = x.shape
    bt = 1024

    return pl.pallas_call(
        _add_body,
        grid=(T // bt, B),
        in_specs=[
            pl.BlockSpec((1, bt, H), lambda t, b: (b, t, 0)),
            pl.BlockSpec((bt, H), lambda t, b: (t, 0)),
        ],
        out_specs=pl.BlockSpec((1, bt, H), lambda t, b: (b, t, 0)),
        out_shape=jax.ShapeDtypeStruct(x.shape, x.dtype),
    )(x, emb[:T])


# speedup vs baseline: 1.7394x; 1.0025x over previous
import jax
import jax.numpy as jnp
from jax.experimental import pallas as pl
from jax.experimental.pallas import tpu as pltpu


def _add_body(x_ref, e_ref, o_ref):
    o_ref[...] = x_ref[...] + e_ref[...]


def kernel(x, emb):
    B, T, H = x.shape
    bt = 512
    bb = 2

    return pl.pallas_call(
        _add_body,
        grid=(T // bt, B // bb),
        in_specs=[
            pl.BlockSpec((bb, bt, H), lambda t, b: (b, t, 0)),
            pl.BlockSpec((bt, H), lambda t, b: (t, 0)),
        ],
        out_specs=pl.BlockSpec((bb, bt, H), lambda t, b: (b, t, 0)),
        out_shape=jax.ShapeDtypeStruct(x.shape, x.dtype),
        compiler_params=pltpu.CompilerParams(
            vmem_limit_bytes=120 * 1024 * 1024,
        ),
    )(x, emb[:T])
